# baseline (speedup 1.0000x reference)
import jax
import jax.numpy as jnp
from jax import lax
from jax.experimental import pallas as pl
from jax.experimental.pallas import tpu as pltpu
from jax.experimental.pallas import tpu_sc as plsc

NUM_CORES = 2
NUM_SUBCORES = 16
LANES = 16
NW = NUM_CORES * NUM_SUBCORES

BATCH = 16384
DIM = 32
PACK = 128 // DIM
BPW = BATCH // NW
QUARTER = BPW // 4
QCHUNKS = QUARTER // LANES


def _bpr_body(uid_hbm, pid_hbm, nid_hbm, uemb_hbm, iemb_hbm,
              outp_hbm, outn_hbm,
              uidx_v, pidx_v, nidx_v, ugidx_v, pgidx_v, ngidx_v,
              urows_v, prows_v, nrows_v,
              outp_v, outn_v, sems):
    wid = lax.axis_index("s") * NUM_CORES + lax.axis_index("c")
    base = wid * BPW

    pltpu.sync_copy(uid_hbm.at[pl.ds(base, BPW)], uidx_v)
    pltpu.sync_copy(pid_hbm.at[pl.ds(base, BPW)], pidx_v)
    pltpu.sync_copy(nid_hbm.at[pl.ds(base, BPW)], nidx_v)

    for j in range(BPW // LANES):
        s = pl.ds(j * LANES, LANES)
        ugidx_v[s] = jax.lax.shift_right_logical(uidx_v[s], 2)
        pgidx_v[s] = jax.lax.shift_right_logical(pidx_v[s], 2)
        ngidx_v[s] = jax.lax.shift_right_logical(nidx_v[s], 2)

    def fire(q, buf):
        s = pl.ds(q * QUARTER, QUARTER)
        return (
            pltpu.async_copy(uemb_hbm.at[ugidx_v.at[s]], urows_v.at[buf],
                             sems.at[buf, 0]),
            pltpu.async_copy(iemb_hbm.at[pgidx_v.at[s]], prows_v.at[buf],
                             sems.at[buf, 1]),
            pltpu.async_copy(iemb_hbm.at[ngidx_v.at[s]], nrows_v.at[buf],
                             sems.at[buf, 2]),
        )

    def compute(q, buf):
        qb = q * QUARTER
        for c in range(QCHUNKS):
            rows = c * LANES + lax.iota(jnp.int32, LANES)
            s = pl.ds(qb + c * LANES, LANES)
            uoff = (uidx_v[s] & (PACK - 1)) * DIM
            poff = (pidx_v[s] & (PACK - 1)) * DIM
            noff = (nidx_v[s] & (PACK - 1)) * DIM
            accp = jnp.zeros((LANES,), jnp.float32)
            accn = jnp.zeros((LANES,), jnp.float32)
            for d in range(DIM):
                u = plsc.load_gather(urows_v.at[buf], [rows, uoff + d])
                p = plsc.load_gather(prows_v.at[buf], [rows, poff + d])
                n = plsc.load_gather(nrows_v.at[buf], [rows, noff + d])
                accp = accp + u * p
                accn = accn + u * n
            outp_v[s] = accp
            outn_v[s] = accn

    copies = fire(0, 0)
    for q in range(4):
        nxt = None
        if q < 3:
            nxt = fire(q + 1, (q + 1) % 2)
        for cp in copies:
            cp.wait()
        compute(q, q % 2)
        copies = nxt

    pltpu.sync_copy(outp_v, outp_hbm.at[pl.ds(base, BPW)])
    pltpu.sync_copy(outn_v, outn_hbm.at[pl.ds(base, BPW)])


def kernel(user_ids, pos_item_ids, neg_item_ids, user_emb, item_emb):
    mesh = plsc.VectorSubcoreMesh(
        core_axis_name="c", subcore_axis_name="s",
        num_cores=NUM_CORES, num_subcores=NUM_SUBCORES)
    out_type = (jax.ShapeDtypeStruct((BATCH,), jnp.float32),
                jax.ShapeDtypeStruct((BATCH,), jnp.float32))
    scratch = [
        pltpu.VMEM((BPW,), jnp.int32),
        pltpu.VMEM((BPW,), jnp.int32),
        pltpu.VMEM((BPW,), jnp.int32),
        pltpu.VMEM((BPW,), jnp.int32),
        pltpu.VMEM((BPW,), jnp.int32),
        pltpu.VMEM((BPW,), jnp.int32),
        pltpu.VMEM((2, QUARTER, 128), jnp.float32),
        pltpu.VMEM((2, QUARTER, 128), jnp.float32),
        pltpu.VMEM((2, QUARTER, 128), jnp.float32),
        pltpu.VMEM((BPW,), jnp.float32),
        pltpu.VMEM((BPW,), jnp.float32),
        pltpu.SemaphoreType.DMA((2, 3)),
    ]
    f = pl.kernel(_bpr_body, out_type=out_type, mesh=mesh,
                  scratch_types=scratch,
                  compiler_params=pltpu.CompilerParams(
                      needs_layout_passes=False,
                      use_tc_tiling_on_sc=True))
    nu, ni = user_emb.shape[0], item_emb.shape[0]
    uemb2 = user_emb.reshape(nu // PACK, DIM * PACK)
    iemb2 = item_emb.reshape(ni // PACK, DIM * PACK)
    return f(user_ids.astype(jnp.int32), pos_item_ids.astype(jnp.int32),
             neg_item_ids.astype(jnp.int32), uemb2, iemb2)
